# baseline (speedup 1.0000x reference)
import functools

import numpy as np
import jax
import jax.numpy as jnp
from jax import lax
from jax.experimental import pallas as pl
from jax.experimental.pallas import tpu as pltpu
from jax.experimental.pallas import tpu_sc as plsc

NHEAD = 16
QUAN = 0.02
_pcr = np.array([-75.2, -75.2, -2.0, 75.2, 75.2, 4.0])
_rngv = _pcr[3:6] - _pcr[0:3]
MAX_LEN = int(float((_rngv ** 2).sum() ** 0.5) // QUAN + 1)

NC, NS = 2, 16
NW = NC * NS
NPTS = 8 * 65536
PPT = NPTS // NW
B = 512
NCH = PPT // B
GSUB = B // 128
ROWS = NPTS // 128
RPB = 65536 // 128

_sel = np.zeros((384, 128), np.float32)
for _l in range(128):
    _sel[3 * _l:3 * _l + 3, _l] = 1.0


def _tc_body(c_ref, scale_ref, q_ref, i1_ref, w2_ref):
    cb = c_ref[...]
    inv = 1.0 / (scale_ref[pl.program_id(0)] * QUAN)
    sq = cb * cb
    ssum = jnp.dot(sq, q_ref[...], preferred_element_type=jnp.float32,
                   precision=lax.Precision.HIGHEST)
    dist = jnp.sqrt(ssum) * inv
    dist = jnp.minimum(dist, jnp.float32(MAX_LEN))
    i1 = dist.astype(jnp.int32)
    w2_ref[...] = dist - i1.astype(jnp.float32)
    i1_ref[...] = jnp.minimum(i1, MAX_LEN - 1)


def _sc_body(pairs_hbm, i1_hbm, w2_hbm, out_hbm,
             i1_v, w2_v, e_v, out_v,
             sem_in0, sem_in1, sem_g0, sem_g1, sem_o0, sem_o1):
    c = lax.axis_index("c")
    s = lax.axis_index("s")
    wid = s * NC + c
    r0 = wid * (PPT // 128)
    sem_in = (sem_in0, sem_in1)
    sem_g = (sem_g0, sem_g1)
    sem_o = (sem_o0, sem_o1)

    def in_descr(g, b):
        rb = r0 + g * GSUB
        pbase = pl.multiple_of(rb * 128, B)
        return (pltpu.make_async_copy(i1_hbm.at[pl.ds(rb, GSUB), :],
                                      i1_v.at[b], sem_in[b]),
                pltpu.make_async_copy(w2_hbm.at[pl.ds(pbase, B)],
                                      w2_v.at[b], sem_in[b]))

    def g_descr(b):
        return [pltpu.make_async_copy(pairs_hbm.at[i1_v.at[b].at[j]],
                                      e_v.at[b].at[pl.ds(j * 128, 128)],
                                      sem_g[b])
                for j in range(GSUB)]

    def out_descr(g, b):
        rb = r0 + g * GSUB
        wbase = pl.multiple_of(rb * 128 * NHEAD, B * NHEAD)
        return pltpu.make_async_copy(out_v.at[b],
                                     out_hbm.at[pl.ds(wbase, B * NHEAD)],
                                     sem_o[b])

    def lerp(b):
        wref = w2_v.at[b]
        eref = e_v.at[b]
        oref = out_v.at[b]

        def ip(t, u):
            q0 = t * 16
            wv = wref[pl.ds(q0, 16)]
            for k in range(16):
                q = q0 + k
                w = wv[k]
                e1 = eref[q, pl.ds(0, 16)]
                e2 = eref[q, pl.ds(16, 16)]
                oref[pl.ds(q * NHEAD, NHEAD)] = e1 + (e2 - e1) * w
            return u

        lax.fori_loop(0, B // 16, ip, 0)

    for d in in_descr(0, 0):
        d.start()
    for d in in_descr(0, 0):
        d.wait()
    for d in g_descr(0):
        d.start()
    for d in in_descr(1, 1):
        d.start()

    def outer(step, carry):
        for bpar in range(2):
            g = step * 2 + bpar
            b, b1 = bpar, 1 - bpar

            @pl.when(g + 1 <= NCH - 1)
            def _():
                for d in in_descr(g + 1, b1):
                    d.wait()
                for d in g_descr(b1):
                    d.start()

            for d in g_descr(b):
                d.wait()
            lerp(b)

            @pl.when(g + 2 <= NCH - 1)
            def _():
                for d in in_descr(g + 2, b):
                    d.start()

            @pl.when(g >= 1)
            def _():
                out_descr(g - 1, b1).wait()

            out_descr(g, b).start()
        return carry

    lax.fori_loop(0, NCH // 2, outer, 0)
    out_descr(NCH - 1, 1).wait()


@jax.jit
def kernel(batch_rel_coords, pred_scale, pos_embed_weight):
    cmat = batch_rel_coords.reshape(ROWS, 384)
    t = pos_embed_weight
    pairs = jnp.concatenate([t, jnp.concatenate([t[1:], t[-1:]], 0)], 1)
    qmat = jnp.asarray(_sel)

    i1, w2 = pl.pallas_call(
        _tc_body,
        grid=(8,),
        in_specs=[
            pl.BlockSpec((RPB, 384), lambda i: (i, 0)),
            pl.BlockSpec(memory_space=pltpu.SMEM),
            pl.BlockSpec((384, 128), lambda i: (0, 0)),
        ],
        out_specs=[
            pl.BlockSpec((RPB, 128), lambda i: (i, 0)),
            pl.BlockSpec((RPB, 128), lambda i: (i, 0)),
        ],
        out_shape=[
            jax.ShapeDtypeStruct((ROWS, 128), jnp.int32),
            jax.ShapeDtypeStruct((ROWS, 128), jnp.float32),
        ],
    )(cmat, pred_scale, qmat)

    mesh = plsc.VectorSubcoreMesh(core_axis_name="c", subcore_axis_name="s")
    run = pl.kernel(
        _sc_body,
        out_type=jax.ShapeDtypeStruct((NPTS * NHEAD,), jnp.float32),
        mesh=mesh,
        compiler_params=pltpu.CompilerParams(use_tc_tiling_on_sc=False),
        scratch_types=[
            pltpu.VMEM((2, GSUB, 128), jnp.int32),
            pltpu.VMEM((2, B), jnp.float32),
            pltpu.VMEM((2, B, 2 * NHEAD), jnp.float32),
            pltpu.VMEM((2, B * NHEAD), jnp.float32),
            pltpu.SemaphoreType.DMA,
            pltpu.SemaphoreType.DMA,
            pltpu.SemaphoreType.DMA,
            pltpu.SemaphoreType.DMA,
            pltpu.SemaphoreType.DMA,
            pltpu.SemaphoreType.DMA,
        ],
    )
    out = run(pairs, i1, w2.reshape(-1))

    one = lax.optimization_barrier(jnp.float32(1.0))
    return out.reshape(8, 65536, NHEAD) * one


# speedup vs baseline: 5.1736x; 5.1736x over previous
import functools

import numpy as np
import jax
import jax.numpy as jnp
from jax import lax
from jax.experimental import pallas as pl
from jax.experimental.pallas import tpu as pltpu
from jax.experimental.pallas import tpu_sc as plsc

NHEAD = 16
QUAN = 0.02
_pcr = np.array([-75.2, -75.2, -2.0, 75.2, 75.2, 4.0])
_rngv = _pcr[3:6] - _pcr[0:3]
MAX_LEN = int(float((_rngv ** 2).sum() ** 0.5) // QUAN + 1)

NC, NS = 2, 16
NW = NC * NS
NPTS = 8 * 65536
PPT = NPTS // NW
B = 512
NCH = PPT // B
GSUB = B // 128
ROWS = NPTS // 128
RPB = 65536 // 128

def _tc_body(c_ref, scale_ref, i1_ref, w2_ref):
    x = c_ref[0, 0:1, :]
    y = c_ref[0, 1:2, :]
    z = c_ref[0, 2:3, :]
    ssum = x * x + y * y + z * z
    inv = 1.0 / (scale_ref[pl.program_id(0)] * QUAN)
    dist = jnp.sqrt(ssum) * inv
    dist = jnp.minimum(dist, jnp.float32(MAX_LEN))
    i1 = dist.astype(jnp.int32)
    w2_ref[...] = (dist - i1.astype(jnp.float32))[None]
    i1_ref[...] = jnp.minimum(i1, MAX_LEN - 1)[None]


def _fmt_body(x_ref, o_ref):
    x = x_ref[...]
    x3 = x.reshape(64, 128, NHEAD)
    xt = jnp.transpose(x3, (0, 2, 1))
    xt2 = jnp.transpose(xt, (1, 0, 2))
    o_ref[...] = xt2.reshape(1, NHEAD, 8192)


def _sc_body(pairs_hbm, i1_hbm, w2_hbm, out_hbm,
             i1_v, w2_v, e_v, out_v,
             sem_in0, sem_in1, sem_g0, sem_g1, sem_o0, sem_o1):
    c = lax.axis_index("c")
    s = lax.axis_index("s")
    wid = s * NC + c
    r0 = wid * (PPT // 128)
    sem_in = (sem_in0, sem_in1)
    sem_g = (sem_g0, sem_g1)
    sem_o = (sem_o0, sem_o1)

    def in_descr(g, b):
        rb = r0 + g * GSUB
        pbase = pl.multiple_of(rb * 128, B)
        return (pltpu.make_async_copy(i1_hbm.at[pl.ds(rb, GSUB), :],
                                      i1_v.at[b], sem_in[b]),
                pltpu.make_async_copy(w2_hbm.at[pl.ds(pbase, B)],
                                      w2_v.at[b], sem_in[b]))

    def g_descr(b):
        return [pltpu.make_async_copy(pairs_hbm.at[i1_v.at[b].at[j]],
                                      e_v.at[b].at[pl.ds(j * 128, 128)],
                                      sem_g[b])
                for j in range(GSUB)]

    def out_descr(g, b):
        rb = r0 + g * GSUB
        wbase = pl.multiple_of(rb * 128 * NHEAD, B * NHEAD)
        return pltpu.make_async_copy(out_v.at[b],
                                     out_hbm.at[pl.ds(wbase, B * NHEAD)],
                                     sem_o[b])

    def lerp(b):
        wref = w2_v.at[b]
        eref = e_v.at[b]
        oref = out_v.at[b]

        def ip(t, u):
            q0 = t * 16
            wv = wref[pl.ds(q0, 16)]
            for k in range(16):
                q = q0 + k
                w = wv[k]
                e1 = eref[q, pl.ds(0, 16)]
                e2 = eref[q, pl.ds(16, 16)]
                oref[pl.ds(q * NHEAD, NHEAD)] = e1 + (e2 - e1) * w
            return u

        lax.fori_loop(0, B // 16, ip, 0)

    for d in in_descr(0, 0):
        d.start()
    for d in in_descr(0, 0):
        d.wait()
    for d in g_descr(0):
        d.start()
    for d in in_descr(1, 1):
        d.start()

    def outer(step, carry):
        for bpar in range(2):
            g = step * 2 + bpar
            b, b1 = bpar, 1 - bpar

            @pl.when(g + 1 <= NCH - 1)
            def _():
                for d in in_descr(g + 1, b1):
                    d.wait()
                for d in g_descr(b1):
                    d.start()

            for d in g_descr(b):
                d.wait()
            lerp(b)

            @pl.when(g + 2 <= NCH - 1)
            def _():
                for d in in_descr(g + 2, b):
                    d.start()

            @pl.when(g >= 1)
            def _():
                out_descr(g - 1, b1).wait()

            out_descr(g, b).start()
        return carry

    lax.fori_loop(0, NCH // 2, outer, 0)
    out_descr(NCH - 1, 1).wait()


@jax.jit
def kernel(batch_rel_coords, pred_scale, pos_embed_weight):
    cmat = jnp.swapaxes(batch_rel_coords, 1, 2)
    t = pos_embed_weight
    pairs = jnp.concatenate([t, jnp.concatenate([t[1:], t[-1:]], 0)], 1)

    i1, w2 = pl.pallas_call(
        _tc_body,
        grid=(8,),
        in_specs=[
            pl.BlockSpec((1, 3, 65536), lambda i: (i, 0, 0)),
            pl.BlockSpec(memory_space=pltpu.SMEM),
        ],
        out_specs=[
            pl.BlockSpec((1, 1, 65536), lambda i: (i, 0, 0)),
            pl.BlockSpec((1, 1, 65536), lambda i: (i, 0, 0)),
        ],
        out_shape=[
            jax.ShapeDtypeStruct((8, 1, 65536), jnp.int32),
            jax.ShapeDtypeStruct((8, 1, 65536), jnp.float32),
        ],
    )(cmat, pred_scale)
    i1 = i1.reshape(ROWS, 128)

    mesh = plsc.VectorSubcoreMesh(core_axis_name="c", subcore_axis_name="s")
    run = pl.kernel(
        _sc_body,
        out_type=jax.ShapeDtypeStruct((NPTS * NHEAD,), jnp.float32),
        mesh=mesh,
        compiler_params=pltpu.CompilerParams(use_tc_tiling_on_sc=False),
        scratch_types=[
            pltpu.VMEM((2, GSUB, 128), jnp.int32),
            pltpu.VMEM((2, B), jnp.float32),
            pltpu.VMEM((2, B, 2 * NHEAD), jnp.float32),
            pltpu.VMEM((2, B * NHEAD), jnp.float32),
            pltpu.SemaphoreType.DMA,
            pltpu.SemaphoreType.DMA,
            pltpu.SemaphoreType.DMA,
            pltpu.SemaphoreType.DMA,
            pltpu.SemaphoreType.DMA,
            pltpu.SemaphoreType.DMA,
        ],
    )
    out = run(pairs, i1, w2.reshape(-1))

    out2d = out.reshape(ROWS, 2048)
    outT = pl.pallas_call(
        _fmt_body,
        grid=(64,),
        in_specs=[pl.BlockSpec((64, 2048), lambda i: (i, 0))],
        out_specs=pl.BlockSpec((1, NHEAD, 8192), lambda i: (i // 8, 0, i % 8)),
        out_shape=jax.ShapeDtypeStruct((8, NHEAD, 65536), jnp.float32),
    )(out2d)
    return jnp.swapaxes(outT, 1, 2)


# speedup vs baseline: 5.2006x; 1.0052x over previous
import functools

import numpy as np
import jax
import jax.numpy as jnp
from jax import lax
from jax.experimental import pallas as pl
from jax.experimental.pallas import tpu as pltpu
from jax.experimental.pallas import tpu_sc as plsc

NHEAD = 16
QUAN = 0.02
_pcr = np.array([-75.2, -75.2, -2.0, 75.2, 75.2, 4.0])
_rngv = _pcr[3:6] - _pcr[0:3]
MAX_LEN = int(float((_rngv ** 2).sum() ** 0.5) // QUAN + 1)

NC, NS = 2, 16
NW = NC * NS
NPTS = 8 * 65536
PPT = NPTS // NW
B = 512
NCH = PPT // B
GSUB = B // 128
ROWS = NPTS // 128
RPB = 65536 // 128

def _tc_body(c_ref, scale_ref, i1_ref, w2_ref):
    x = c_ref[0, 0:1, :]
    y = c_ref[0, 1:2, :]
    z = c_ref[0, 2:3, :]
    ssum = x * x + y * y + z * z
    inv = 1.0 / (scale_ref[pl.program_id(0)] * QUAN)
    dist = jnp.sqrt(ssum) * inv
    dist = jnp.minimum(dist, jnp.float32(MAX_LEN))
    i1 = dist.astype(jnp.int32)
    w2_ref[...] = (dist - i1.astype(jnp.float32))[None]
    i1_ref[...] = jnp.minimum(i1, MAX_LEN - 1)[None]


def _fmt_body(x_ref, o_ref):
    x = x_ref[...]
    x3 = x.reshape(64, 128, NHEAD)
    xt = jnp.transpose(x3, (0, 2, 1))
    xt2 = jnp.transpose(xt, (1, 0, 2))
    o_ref[...] = xt2.reshape(1, NHEAD, 8192)


def _sc_body(pairs_hbm, i1_hbm, w2_hbm, out_hbm,
             i1_v, w2_v, e_v, out_v,
             sem_in0, sem_in1, sem_g0, sem_g1, sem_o0, sem_o1):
    c = lax.axis_index("c")
    s = lax.axis_index("s")
    wid = s * NC + c
    r0 = wid * (PPT // 128)
    sem_in = (sem_in0, sem_in1)
    sem_g = (sem_g0, sem_g1)
    sem_o = (sem_o0, sem_o1)

    def in_descr(g, b):
        rb = r0 + g * GSUB
        pbase = pl.multiple_of(rb * 128, B)
        return (pltpu.make_async_copy(i1_hbm.at[pl.ds(rb, GSUB), :],
                                      i1_v.at[b], sem_in[b]),
                pltpu.make_async_copy(w2_hbm.at[pl.ds(pbase, B)],
                                      w2_v.at[b], sem_in[b]))

    def g_descr(b):
        return [pltpu.make_async_copy(pairs_hbm.at[i1_v.at[b].at[j]],
                                      e_v.at[b].at[pl.ds(j * 128, 128)],
                                      sem_g[b])
                for j in range(GSUB)]

    def out_descr(g, b):
        rb = r0 + g * GSUB
        wbase = pl.multiple_of(rb * 128 * NHEAD, B * NHEAD)
        return pltpu.make_async_copy(out_v.at[b],
                                     out_hbm.at[pl.ds(wbase, B * NHEAD)],
                                     sem_o[b])

    def lerp(b):
        wref = w2_v.at[b]
        eref = e_v.at[b]
        oref = out_v.at[b]

        def ip(t, u):
            q0 = t * 16
            we0 = q0 * 32
            wo0 = q0 * NHEAD
            wv = wref[pl.ds(q0, 16)]
            for k in range(16):
                w = wv.at[jnp.full((16,), k, jnp.int32)].get(
                    mode="promise_in_bounds")
                e1 = eref[q0 + k, pl.ds(0, 16)]
                e2 = eref[q0 + k, pl.ds(16, 16)]
                oref[pl.ds(wo0 + NHEAD * k, NHEAD)] = e1 + (e2 - e1) * w
            return u

        lax.fori_loop(0, B // 16, ip, 0)

    for d in in_descr(0, 0):
        d.start()
    for d in in_descr(0, 0):
        d.wait()
    for d in g_descr(0):
        d.start()
    for d in in_descr(1, 1):
        d.start()

    def outer(step, carry):
        for bpar in range(2):
            g = step * 2 + bpar
            b, b1 = bpar, 1 - bpar

            @pl.when(g + 1 <= NCH - 1)
            def _():
                for d in in_descr(g + 1, b1):
                    d.wait()
                for d in g_descr(b1):
                    d.start()

            for d in g_descr(b):
                d.wait()
            lerp(b)

            @pl.when(g + 2 <= NCH - 1)
            def _():
                for d in in_descr(g + 2, b):
                    d.start()

            @pl.when(g >= 1)
            def _():
                out_descr(g - 1, b1).wait()

            out_descr(g, b).start()
        return carry

    lax.fori_loop(0, NCH // 2, outer, 0)
    out_descr(NCH - 1, 1).wait()


@jax.jit
def kernel(batch_rel_coords, pred_scale, pos_embed_weight):
    cmat = jnp.swapaxes(batch_rel_coords, 1, 2)
    t = pos_embed_weight
    pairs = jnp.concatenate([t, jnp.concatenate([t[1:], t[-1:]], 0)], 1)

    i1, w2 = pl.pallas_call(
        _tc_body,
        grid=(8,),
        in_specs=[
            pl.BlockSpec((1, 3, 65536), lambda i: (i, 0, 0)),
            pl.BlockSpec(memory_space=pltpu.SMEM),
        ],
        out_specs=[
            pl.BlockSpec((1, 1, 65536), lambda i: (i, 0, 0)),
            pl.BlockSpec((1, 1, 65536), lambda i: (i, 0, 0)),
        ],
        out_shape=[
            jax.ShapeDtypeStruct((8, 1, 65536), jnp.int32),
            jax.ShapeDtypeStruct((8, 1, 65536), jnp.float32),
        ],
    )(cmat, pred_scale)
    i1 = i1.reshape(ROWS, 128)

    mesh = plsc.VectorSubcoreMesh(core_axis_name="c", subcore_axis_name="s")
    run = pl.kernel(
        _sc_body,
        out_type=jax.ShapeDtypeStruct((NPTS * NHEAD,), jnp.float32),
        mesh=mesh,
        compiler_params=pltpu.CompilerParams(use_tc_tiling_on_sc=False),
        scratch_types=[
            pltpu.VMEM((2, GSUB, 128), jnp.int32),
            pltpu.VMEM((2, B), jnp.float32),
            pltpu.VMEM((2, B, 2 * NHEAD), jnp.float32),
            pltpu.VMEM((2, B * NHEAD), jnp.float32),
            pltpu.SemaphoreType.DMA,
            pltpu.SemaphoreType.DMA,
            pltpu.SemaphoreType.DMA,
            pltpu.SemaphoreType.DMA,
            pltpu.SemaphoreType.DMA,
            pltpu.SemaphoreType.DMA,
        ],
    )
    out = run(pairs, i1, w2.reshape(-1))

    out2d = out.reshape(ROWS, 2048)
    outT = pl.pallas_call(
        _fmt_body,
        grid=(64,),
        in_specs=[pl.BlockSpec((64, 2048), lambda i: (i, 0))],
        out_specs=pl.BlockSpec((1, NHEAD, 8192), lambda i: (i // 8, 0, i % 8)),
        out_shape=jax.ShapeDtypeStruct((8, NHEAD, 65536), jnp.float32),
    )(out2d)
    return jnp.swapaxes(outT, 1, 2)


# speedup vs baseline: 5.3704x; 1.0326x over previous
import functools

import numpy as np
import jax
import jax.numpy as jnp
from jax import lax
from jax.experimental import pallas as pl
from jax.experimental.pallas import tpu as pltpu
from jax.experimental.pallas import tpu_sc as plsc

NHEAD = 16
QUAN = 0.02
_pcr = np.array([-75.2, -75.2, -2.0, 75.2, 75.2, 4.0])
_rngv = _pcr[3:6] - _pcr[0:3]
MAX_LEN = int(float((_rngv ** 2).sum() ** 0.5) // QUAN + 1)

NC, NS = 2, 16
NW = NC * NS
NPTS = 8 * 65536
PPT = NPTS // NW
B = 1024
NCH = PPT // B
GSUB = B // 128
ROWS = NPTS // 128
RPB = 65536 // 128

def _tc_body(c_ref, scale_ref, i1_ref, w2_ref):
    x = c_ref[0]
    y = c_ref[1]
    z = c_ref[2]
    ssum = x * x + y * y + z * z
    rid = lax.broadcasted_iota(jnp.int32, (8, 1), 0)
    inv = jnp.zeros((8, 1), jnp.float32)
    for b in range(8):
        inv = jnp.where(rid == b, 1.0 / (scale_ref[b] * QUAN), inv)
    dist = jnp.sqrt(ssum) * inv
    dist = jnp.minimum(dist, jnp.float32(MAX_LEN))
    i1 = dist.astype(jnp.int32)
    w2_ref[...] = dist - i1.astype(jnp.float32)
    i1_ref[...] = jnp.minimum(i1, MAX_LEN - 1)


def _fmt_body(x_ref, o_ref):
    x = x_ref[...]
    x3 = x.reshape(64, 128, NHEAD)
    xt2 = jnp.transpose(x3, (2, 0, 1))
    o_ref[...] = xt2.reshape(1, NHEAD, 8192)


def _sc_body(pairs_hbm, i1_hbm, w2_hbm, out_hbm,
             i1_v, w2_v, e_v, out_v,
             sem_in0, sem_in1, sem_g0, sem_g1, sem_o0, sem_o1):
    c = lax.axis_index("c")
    s = lax.axis_index("s")
    wid = s * NC + c
    r0 = wid * (PPT // 128)
    sem_in = (sem_in0, sem_in1)
    sem_g = (sem_g0, sem_g1)
    sem_o = (sem_o0, sem_o1)

    def in_descr(g, b):
        rb = r0 + g * GSUB
        pbase = pl.multiple_of(rb * 128, B)
        return (pltpu.make_async_copy(i1_hbm.at[pl.ds(rb, GSUB), :],
                                      i1_v.at[b], sem_in[b]),
                pltpu.make_async_copy(w2_hbm.at[pl.ds(pbase, B)],
                                      w2_v.at[b], sem_in[b]))

    def g_descr(b):
        return [pltpu.make_async_copy(pairs_hbm.at[i1_v.at[b].at[j]],
                                      e_v.at[b].at[pl.ds(j * 128, 128)],
                                      sem_g[b])
                for j in range(GSUB)]

    def out_descr(g, b):
        rb = r0 + g * GSUB
        wbase = pl.multiple_of(rb * 128 * NHEAD, B * NHEAD)
        return pltpu.make_async_copy(out_v.at[b],
                                     out_hbm.at[pl.ds(wbase, B * NHEAD)],
                                     sem_o[b])

    def lerp(b):
        wref = w2_v.at[b]
        eref = e_v.at[b]
        oref = out_v.at[b]

        def ip(t, u):
            q0 = t * 16
            we0 = q0 * 32
            wo0 = q0 * NHEAD
            wv = wref[pl.ds(q0, 16)]
            for k in range(16):
                w = wv.at[jnp.full((16,), k, jnp.int32)].get(
                    mode="promise_in_bounds")
                e1 = eref[q0 + k, pl.ds(0, 16)]
                e2 = eref[q0 + k, pl.ds(16, 16)]
                oref[pl.ds(wo0 + NHEAD * k, NHEAD)] = e1 + (e2 - e1) * w
            return u

        lax.fori_loop(0, B // 16, ip, 0)

    for d in in_descr(0, 0):
        d.start()
    for d in in_descr(0, 0):
        d.wait()
    for d in g_descr(0):
        d.start()
    for d in in_descr(1, 1):
        d.start()

    def outer(step, carry):
        for bpar in range(2):
            g = step * 2 + bpar
            b, b1 = bpar, 1 - bpar

            @pl.when(g + 1 <= NCH - 1)
            def _():
                for d in in_descr(g + 1, b1):
                    d.wait()
                for d in g_descr(b1):
                    d.start()

            for d in g_descr(b):
                d.wait()
            lerp(b)

            @pl.when(g + 2 <= NCH - 1)
            def _():
                for d in in_descr(g + 2, b):
                    d.start()

            @pl.when(g >= 1)
            def _():
                out_descr(g - 1, b1).wait()

            out_descr(g, b).start()
        return carry

    lax.fori_loop(0, NCH // 2, outer, 0)
    out_descr(NCH - 1, 1).wait()


@jax.jit
def kernel(batch_rel_coords, pred_scale, pos_embed_weight):
    cmat = jnp.transpose(batch_rel_coords, (2, 0, 1))
    t = pos_embed_weight
    pairs = jnp.concatenate([t, jnp.concatenate([t[1:], t[-1:]], 0)], 1)

    i1, w2 = pl.pallas_call(
        _tc_body,
        in_specs=[
            pl.BlockSpec((3, 8, 65536), lambda: (0, 0, 0)),
            pl.BlockSpec(memory_space=pltpu.SMEM),
        ],
        out_specs=[
            pl.BlockSpec((8, 65536), lambda: (0, 0)),
            pl.BlockSpec((8, 65536), lambda: (0, 0)),
        ],
        out_shape=[
            jax.ShapeDtypeStruct((8, 65536), jnp.int32),
            jax.ShapeDtypeStruct((8, 65536), jnp.float32),
        ],
    )(cmat, pred_scale)
    i1 = i1.reshape(ROWS, 128)

    mesh = plsc.VectorSubcoreMesh(core_axis_name="c", subcore_axis_name="s")
    run = pl.kernel(
        _sc_body,
        out_type=jax.ShapeDtypeStruct((NPTS * NHEAD,), jnp.float32),
        mesh=mesh,
        compiler_params=pltpu.CompilerParams(use_tc_tiling_on_sc=False),
        scratch_types=[
            pltpu.VMEM((2, GSUB, 128), jnp.int32),
            pltpu.VMEM((2, B), jnp.float32),
            pltpu.VMEM((2, B, 2 * NHEAD), jnp.float32),
            pltpu.VMEM((2, B * NHEAD), jnp.float32),
            pltpu.SemaphoreType.DMA,
            pltpu.SemaphoreType.DMA,
            pltpu.SemaphoreType.DMA,
            pltpu.SemaphoreType.DMA,
            pltpu.SemaphoreType.DMA,
            pltpu.SemaphoreType.DMA,
        ],
    )
    out = run(pairs, i1, w2.reshape(-1))

    out2d = out.reshape(ROWS, 2048)
    outT = pl.pallas_call(
        _fmt_body,
        grid=(64,),
        in_specs=[pl.BlockSpec((64, 2048), lambda i: (i, 0))],
        out_specs=pl.BlockSpec((1, NHEAD, 8192), lambda i: (i // 8, 0, i % 8)),
        out_shape=jax.ShapeDtypeStruct((8, NHEAD, 65536), jnp.float32),
    )(out2d)
    return jnp.swapaxes(outT, 1, 2)


# speedup vs baseline: 5.5255x; 1.0289x over previous
import functools

import numpy as np
import jax
import jax.numpy as jnp
from jax import lax
from jax.experimental import pallas as pl
from jax.experimental.pallas import tpu as pltpu
from jax.experimental.pallas import tpu_sc as plsc

NHEAD = 16
QUAN = 0.02
_pcr = np.array([-75.2, -75.2, -2.0, 75.2, 75.2, 4.0])
_rngv = _pcr[3:6] - _pcr[0:3]
MAX_LEN = int(float((_rngv ** 2).sum() ** 0.5) // QUAN + 1)

NC, NS = 2, 16
NW = NC * NS
NPTS = 8 * 65536
PPT = NPTS // NW
B = 1024
NCH = PPT // B
GSUB = B // 128
ROWS = NPTS // 128
RPB = 65536 // 128

def _tc_body(c_ref, scale_ref, i1_ref, w2_ref):
    x = c_ref[0]
    y = c_ref[1]
    z = c_ref[2]
    ssum = x * x + y * y + z * z
    rid = lax.broadcasted_iota(jnp.int32, (8, 1), 0)
    inv = jnp.zeros((8, 1), jnp.float32)
    for b in range(8):
        inv = jnp.where(rid == b, 1.0 / (scale_ref[b] * QUAN), inv)
    dist = jnp.sqrt(ssum) * inv
    dist = jnp.minimum(dist, jnp.float32(MAX_LEN))
    i1 = dist.astype(jnp.int32)
    w2_ref[...] = dist - i1.astype(jnp.float32)
    i1_ref[...] = jnp.minimum(i1, MAX_LEN - 1)


def _fmt_body(x_ref, o_ref):
    x = x_ref[...]
    x3 = x.reshape(64, 128, NHEAD)
    xt2 = jnp.transpose(x3, (2, 0, 1))
    o_ref[...] = xt2.reshape(1, NHEAD, 8192)


def _sc_body(pairs_hbm, i1_hbm, w2_hbm, out_hbm,
             i1_v, w2_v, e_v, out_v,
             sem_in0, sem_in1, sem_g0, sem_g1, sem_o0, sem_o1):
    c = lax.axis_index("c")
    s = lax.axis_index("s")
    wid = s * NC + c
    r0 = wid * (PPT // 128)
    sem_in = (sem_in0, sem_in1)
    sem_g = (sem_g0, sem_g1)
    sem_o = (sem_o0, sem_o1)

    def in_descr(g, b):
        rb = r0 + g * GSUB
        pbase = pl.multiple_of(rb * 128, B)
        return (pltpu.make_async_copy(i1_hbm.at[pl.ds(rb, GSUB), :],
                                      i1_v.at[b], sem_in[b]),
                pltpu.make_async_copy(w2_hbm.at[pl.ds(pbase, B)],
                                      w2_v.at[b], sem_in[b]))

    def g_descr(b):
        return [pltpu.make_async_copy(pairs_hbm.at[i1_v.at[b].at[j]],
                                      e_v.at[b].at[pl.ds(j * 128, 128)],
                                      sem_g[b])
                for j in range(GSUB)]

    def out_descr(g, b):
        rb = r0 + g * GSUB
        wbase = pl.multiple_of(rb * 128 * NHEAD, B * NHEAD)
        return pltpu.make_async_copy(out_v.at[b],
                                     out_hbm.at[pl.ds(wbase, B * NHEAD)],
                                     sem_o[b])

    def lerp(b):
        wref = w2_v.at[b]
        eref = e_v.at[b]
        oref = out_v.at[b]

        @plsc.parallel_loop(0, B // 16, 1)
        def ip(t):
            q0 = t * 16
            wo0 = q0 * NHEAD
            wv = wref[pl.ds(q0, 16)]
            for k in range(16):
                w = wv.at[jnp.full((16,), k, jnp.int32)].get(
                    mode="promise_in_bounds")
                e1 = eref[q0 + k, pl.ds(0, 16)]
                e2 = eref[q0 + k, pl.ds(16, 16)]
                oref[pl.ds(wo0 + NHEAD * k, NHEAD)] = e1 + (e2 - e1) * w

    for d in in_descr(0, 0):
        d.start()
    for d in in_descr(0, 0):
        d.wait()
    for d in g_descr(0):
        d.start()
    for d in in_descr(1, 1):
        d.start()

    def outer(step, carry):
        for bpar in range(2):
            g = step * 2 + bpar
            b, b1 = bpar, 1 - bpar

            @pl.when(g + 1 <= NCH - 1)
            def _():
                for d in in_descr(g + 1, b1):
                    d.wait()
                for d in g_descr(b1):
                    d.start()

            for d in g_descr(b):
                d.wait()
            lerp(b)

            @pl.when(g + 2 <= NCH - 1)
            def _():
                for d in in_descr(g + 2, b):
                    d.start()

            @pl.when(g >= 1)
            def _():
                out_descr(g - 1, b1).wait()

            out_descr(g, b).start()
        return carry

    lax.fori_loop(0, NCH // 2, outer, 0)
    out_descr(NCH - 1, 1).wait()


@jax.jit
def kernel(batch_rel_coords, pred_scale, pos_embed_weight):
    cmat = jnp.transpose(batch_rel_coords, (2, 0, 1))
    t = pos_embed_weight
    pairs = jnp.concatenate([t, jnp.concatenate([t[1:], t[-1:]], 0)], 1)

    i1, w2 = pl.pallas_call(
        _tc_body,
        in_specs=[
            pl.BlockSpec((3, 8, 65536), lambda: (0, 0, 0)),
            pl.BlockSpec(memory_space=pltpu.SMEM),
        ],
        out_specs=[
            pl.BlockSpec((8, 65536), lambda: (0, 0)),
            pl.BlockSpec((8, 65536), lambda: (0, 0)),
        ],
        out_shape=[
            jax.ShapeDtypeStruct((8, 65536), jnp.int32),
            jax.ShapeDtypeStruct((8, 65536), jnp.float32),
        ],
    )(cmat, pred_scale)
    i1 = i1.reshape(ROWS, 128)

    mesh = plsc.VectorSubcoreMesh(core_axis_name="c", subcore_axis_name="s")
    run = pl.kernel(
        _sc_body,
        out_type=jax.ShapeDtypeStruct((NPTS * NHEAD,), jnp.float32),
        mesh=mesh,
        compiler_params=pltpu.CompilerParams(use_tc_tiling_on_sc=False),
        scratch_types=[
            pltpu.VMEM((2, GSUB, 128), jnp.int32),
            pltpu.VMEM((2, B), jnp.float32),
            pltpu.VMEM((2, B, 2 * NHEAD), jnp.float32),
            pltpu.VMEM((2, B * NHEAD), jnp.float32),
            pltpu.SemaphoreType.DMA,
            pltpu.SemaphoreType.DMA,
            pltpu.SemaphoreType.DMA,
            pltpu.SemaphoreType.DMA,
            pltpu.SemaphoreType.DMA,
            pltpu.SemaphoreType.DMA,
        ],
    )
    out = run(pairs, i1, w2.reshape(-1))

    out2d = out.reshape(ROWS, 2048)
    outT = pl.pallas_call(
        _fmt_body,
        grid=(64,),
        in_specs=[pl.BlockSpec((64, 2048), lambda i: (i, 0))],
        out_specs=pl.BlockSpec((1, NHEAD, 8192), lambda i: (i // 8, 0, i % 8)),
        out_shape=jax.ShapeDtypeStruct((8, NHEAD, 65536), jnp.float32),
    )(out2d)
    return jnp.swapaxes(outT, 1, 2)


# speedup vs baseline: 5.5363x; 1.0020x over previous
import functools

import numpy as np
import jax
import jax.numpy as jnp
from jax import lax
from jax.experimental import pallas as pl
from jax.experimental.pallas import tpu as pltpu
from jax.experimental.pallas import tpu_sc as plsc

NHEAD = 16
QUAN = 0.02
_pcr = np.array([-75.2, -75.2, -2.0, 75.2, 75.2, 4.0])
_rngv = _pcr[3:6] - _pcr[0:3]
MAX_LEN = int(float((_rngv ** 2).sum() ** 0.5) // QUAN + 1)

NC, NS = 2, 16
NW = NC * NS
NPTS = 8 * 65536
PPT = NPTS // NW
B = 1024
NCH = PPT // B
GSUB = B // 128
ROWS = NPTS // 128
RPB = 65536 // 128

def _tc_body(c_ref, scale_ref, i1_ref, w2_ref):
    x = c_ref[0]
    y = c_ref[1]
    z = c_ref[2]
    ssum = x * x + y * y + z * z
    rid = lax.broadcasted_iota(jnp.int32, (8, 1), 0)
    inv = jnp.zeros((8, 1), jnp.float32)
    for b in range(8):
        inv = jnp.where(rid == b, 1.0 / (scale_ref[b] * QUAN), inv)
    dist = jnp.sqrt(ssum) * inv
    dist = jnp.minimum(dist, jnp.float32(MAX_LEN))
    i1 = dist.astype(jnp.int32)
    w2_ref[...] = dist - i1.astype(jnp.float32)
    i1_ref[...] = jnp.minimum(i1, MAX_LEN - 1)


def _fmt_body(x_ref, o_ref):
    x = x_ref[...]
    x3 = x.reshape(64, 128, NHEAD)
    xt2 = jnp.transpose(x3, (2, 0, 1))
    o_ref[...] = xt2.reshape(1, NHEAD, 8192)


def _sc_body(pairs_hbm, i1_hbm, w2_hbm, out_hbm,
             i1_v, w2_v, e_v, out_v,
             sem_in0, sem_in1, sem_g0, sem_g1, sem_o0, sem_o1):
    c = lax.axis_index("c")
    s = lax.axis_index("s")
    wid = s * NC + c
    r0 = wid * (PPT // 128)
    sem_in = (sem_in0, sem_in1)
    sem_g = (sem_g0, sem_g1)
    sem_o = (sem_o0, sem_o1)

    def in_descr(g, b):
        rb = r0 + g * GSUB
        pbase = pl.multiple_of(rb * 128, B)
        return (pltpu.make_async_copy(i1_hbm.at[pl.ds(rb, GSUB), :],
                                      i1_v.at[b], sem_in[b]),
                pltpu.make_async_copy(w2_hbm.at[pl.ds(pbase, B)],
                                      w2_v.at[b], sem_in[b]))

    def g_descr(b):
        return [pltpu.make_async_copy(pairs_hbm.at[i1_v.at[b].at[j]],
                                      e_v.at[b].at[pl.ds(j * 128, 128)],
                                      sem_g[b])
                for j in range(GSUB)]

    def out_descr(g, b):
        rb = r0 + g * GSUB
        wbase = pl.multiple_of(rb * 128 * NHEAD, B * NHEAD)
        return pltpu.make_async_copy(out_v.at[b],
                                     out_hbm.at[pl.ds(wbase, B * NHEAD)],
                                     sem_o[b])

    def lerp(b):
        wref = w2_v.at[b]
        eref = e_v.at[b]
        oref = out_v.at[b]

        @plsc.parallel_loop(0, B // 16, 1, unroll=2)
        def ip(t):
            q0 = t * 16
            wo0 = q0 * NHEAD
            wv = wref[pl.ds(q0, 16)]
            for k in range(16):
                w = wv.at[jnp.full((16,), k, jnp.int32)].get(
                    mode="promise_in_bounds")
                e1 = eref[q0 + k, pl.ds(0, 16)]
                e2 = eref[q0 + k, pl.ds(16, 16)]
                oref[pl.ds(wo0 + NHEAD * k, NHEAD)] = e1 + (e2 - e1) * w

    for d in in_descr(0, 0):
        d.start()
    for d in in_descr(0, 0):
        d.wait()
    for d in g_descr(0):
        d.start()
    for d in in_descr(1, 1):
        d.start()

    def outer(step, carry):
        for bpar in range(2):
            g = step * 2 + bpar
            b, b1 = bpar, 1 - bpar

            @pl.when(g + 1 <= NCH - 1)
            def _():
                for d in in_descr(g + 1, b1):
                    d.wait()
                for d in g_descr(b1):
                    d.start()

            for d in g_descr(b):
                d.wait()
            lerp(b)

            @pl.when(g + 2 <= NCH - 1)
            def _():
                for d in in_descr(g + 2, b):
                    d.start()

            @pl.when(g >= 1)
            def _():
                out_descr(g - 1, b1).wait()

            out_descr(g, b).start()
        return carry

    lax.fori_loop(0, NCH // 2, outer, 0)
    out_descr(NCH - 1, 1).wait()


@jax.jit
def kernel(batch_rel_coords, pred_scale, pos_embed_weight):
    cmat = jnp.transpose(batch_rel_coords, (2, 0, 1))
    t = pos_embed_weight
    pairs = jnp.concatenate([t, jnp.concatenate([t[1:], t[-1:]], 0)], 1)

    i1, w2 = pl.pallas_call(
        _tc_body,
        in_specs=[
            pl.BlockSpec((3, 8, 65536), lambda: (0, 0, 0)),
            pl.BlockSpec(memory_space=pltpu.SMEM),
        ],
        out_specs=[
            pl.BlockSpec((8, 65536), lambda: (0, 0)),
            pl.BlockSpec((8, 65536), lambda: (0, 0)),
        ],
        out_shape=[
            jax.ShapeDtypeStruct((8, 65536), jnp.int32),
            jax.ShapeDtypeStruct((8, 65536), jnp.float32),
        ],
    )(cmat, pred_scale)
    i1 = i1.reshape(ROWS, 128)

    mesh = plsc.VectorSubcoreMesh(core_axis_name="c", subcore_axis_name="s")
    run = pl.kernel(
        _sc_body,
        out_type=jax.ShapeDtypeStruct((NPTS * NHEAD,), jnp.float32),
        mesh=mesh,
        compiler_params=pltpu.CompilerParams(use_tc_tiling_on_sc=False),
        scratch_types=[
            pltpu.VMEM((2, GSUB, 128), jnp.int32),
            pltpu.VMEM((2, B), jnp.float32),
            pltpu.VMEM((2, B, 2 * NHEAD), jnp.float32),
            pltpu.VMEM((2, B * NHEAD), jnp.float32),
            pltpu.SemaphoreType.DMA,
            pltpu.SemaphoreType.DMA,
            pltpu.SemaphoreType.DMA,
            pltpu.SemaphoreType.DMA,
            pltpu.SemaphoreType.DMA,
            pltpu.SemaphoreType.DMA,
        ],
    )
    out = run(pairs, i1, w2.reshape(-1))

    out2d = out.reshape(ROWS, 2048)
    outT = pl.pallas_call(
        _fmt_body,
        grid=(64,),
        in_specs=[pl.BlockSpec((64, 2048), lambda i: (i, 0))],
        out_specs=pl.BlockSpec((1, NHEAD, 8192), lambda i: (i // 8, 0, i % 8)),
        out_shape=jax.ShapeDtypeStruct((8, NHEAD, 65536), jnp.float32),
    )(out2d)
    return jnp.swapaxes(outT, 1, 2)
